# baseline (device time: 67930 ns/iter reference)
import os

import jax
import jax.numpy as jnp
from jax import lax
from jax.experimental import pallas as pl
from jax.experimental.pallas import tpu as pltpu

W = 8
LOG_W = 3
PLAN = [(11, 0), (12, 1), (12, 0), (13, 2), (13, 1), (13, 0)]
TAIL_AFTER = {0: 11, 2: 12, 5: 13}
N_EXCH = len(PLAN)
NO_EXCH = bool(os.environ.get("SORT_NO_EXCH"))


def _cmpex(v, s, blk, offset):
    m = v.shape[0]
    g = lax.broadcasted_iota(jnp.int32, v.shape, 0) + offset
    partner_above = (g & s) == 0
    down = jnp.concatenate([v[s:], v[:s]], axis=0)
    up = jnp.concatenate([v[m - s:], v[:m - s]], axis=0)
    partner = jnp.where(partner_above, down, up)
    asc = (g & blk) == 0
    take_min = partner_above == asc
    return jnp.where(take_min, jnp.minimum(v, partner), jnp.maximum(v, partner))


def kernel(x):
    m_per, n = x.shape
    n_half = n // 2
    log_m_per = m_per.bit_length() - 1

    def body(x_ref, out_ref, sbufs, rbufs, send_sems, recv_sems):
        my = lax.axis_index("i")
        offset = my * m_per

        barrier_sem = pltpu.get_barrier_semaphore()
        for t in range(LOG_W):
            pl.semaphore_signal(
                barrier_sem, inc=1,
                device_id=(my ^ (1 << t),),
                device_id_type=pl.DeviceIdType.MESH,
            )
        pl.semaphore_wait(barrier_sem, LOG_W)

        def local_sort(v):
            for k in range(1, log_m_per + 1):
                for j in range(k - 1, -1, -1):
                    v = _cmpex(v, 1 << j, 1 << k, offset)
            return v

        def local_tail(v, k):
            for j in range(log_m_per - 1, -1, -1):
                v = _cmpex(v, 1 << j, 1 << k, offset)
            return v

        def start_exch(grp, e, v):
            _, t = PLAN[e]
            slot = grp * N_EXCH + e
            sbufs[grp] = v
            rdma = pltpu.make_async_remote_copy(
                src_ref=sbufs.at[grp],
                dst_ref=rbufs.at[slot],
                send_sem=send_sems.at[slot],
                recv_sem=recv_sems.at[slot],
                device_id=(my ^ (1 << t),),
                device_id_type=pl.DeviceIdType.MESH,
            )
            if not NO_EXCH:
                rdma.start()
            return rdma

        def finish_exch(grp, e, v, rdma):
            if not NO_EXCH:
                rdma.wait()
            k, t = PLAN[e]
            other = rbufs[grp * N_EXCH + e]
            asc = (offset & (1 << k)) == 0
            partner_above = (offset & (m_per << t)) == 0
            take_min = partner_above == asc
            v = jnp.where(
                take_min, jnp.minimum(v, other), jnp.maximum(v, other)
            )
            if e in TAIL_AFTER:
                v = local_tail(v, TAIL_AFTER[e])
            return v

        vA = x_ref[:, :n_half].astype(jnp.bfloat16)
        vB = x_ref[:, n_half:].astype(jnp.bfloat16)

        vA = local_sort(vA)
        rA = start_exch(0, 0, vA)
        vB = local_sort(vB)
        rB = start_exch(1, 0, vB)
        for e in range(N_EXCH):
            vA = finish_exch(0, e, vA, rA)
            if e + 1 < N_EXCH:
                rA = start_exch(0, e + 1, vA)
            vB = finish_exch(1, e, vB, rB)
            if e + 1 < N_EXCH:
                rB = start_exch(1, e + 1, vB)

        out_ref[:, :n_half] = vA.astype(out_ref.dtype)
        out_ref[:, n_half:] = vB.astype(out_ref.dtype)

    return pl.pallas_call(
        body,
        out_shape=jax.ShapeDtypeStruct((m_per, n), x.dtype),
        in_specs=[pl.BlockSpec(memory_space=pltpu.VMEM)],
        out_specs=pl.BlockSpec(memory_space=pltpu.VMEM),
        scratch_shapes=[
            pltpu.VMEM((2, m_per, n_half), jnp.bfloat16),
            pltpu.VMEM((2 * N_EXCH, m_per, n_half), jnp.bfloat16),
            pltpu.SemaphoreType.DMA((2 * N_EXCH,)),
            pltpu.SemaphoreType.DMA((2 * N_EXCH,)),
        ],
        compiler_params=pltpu.CompilerParams(
            collective_id=0,
            vmem_limit_bytes=100 * 1024 * 1024,
        ),
    )(x)


# device time: 60509 ns/iter; 1.1226x vs baseline; 1.1226x over previous
import os

import jax
import jax.numpy as jnp
from jax import lax
from jax.experimental import pallas as pl
from jax.experimental.pallas import tpu as pltpu

W = 8
LOG_W = 3
PLAN = [(11, 0), (12, 1), (12, 0), (13, 2), (13, 1), (13, 0)]
TAIL_AFTER = {0: 11, 2: 12, 5: 13}
N_EXCH = len(PLAN)
NO_EXCH = bool(os.environ.get("SORT_NO_EXCH"))


def _cmpex(v, s, blk, offset):
    m, n = v.shape
    if s >= 16:
        grp = m // (2 * s)
        y = v.reshape(grp, 2, s, n)
        a, b = y[:, 0], y[:, 1]
        lo, hi = jnp.minimum(a, b), jnp.maximum(a, b)
        g0 = lax.broadcasted_iota(jnp.int32, (grp, 1, 1), 0) * (2 * s) + offset
        asc = (g0 & blk) == 0
        first = jnp.where(asc, lo, hi)
        second = jnp.where(asc, hi, lo)
        out = jnp.concatenate([first[:, None], second[:, None]], axis=1)
        return out.reshape(m, n)
    g = lax.broadcasted_iota(jnp.int32, v.shape, 0) + offset
    partner_above = (g & s) == 0
    down = jnp.concatenate([v[s:], v[:s]], axis=0)
    up = jnp.concatenate([v[m - s:], v[:m - s]], axis=0)
    partner = jnp.where(partner_above, down, up)
    asc = (g & blk) == 0
    take_min = partner_above == asc
    return jnp.where(take_min, jnp.minimum(v, partner), jnp.maximum(v, partner))


def kernel(x):
    m_per, n = x.shape
    n_half = n // 2
    log_m_per = m_per.bit_length() - 1

    def body(x_ref, out_ref, sbufs, rbufs, send_sems, recv_sems):
        my = lax.axis_index("i")
        offset = my * m_per

        barrier_sem = pltpu.get_barrier_semaphore()
        for t in range(LOG_W):
            pl.semaphore_signal(
                barrier_sem, inc=1,
                device_id=(my ^ (1 << t),),
                device_id_type=pl.DeviceIdType.MESH,
            )
        pl.semaphore_wait(barrier_sem, LOG_W)

        def local_sort(v):
            for k in range(1, log_m_per + 1):
                for j in range(k - 1, -1, -1):
                    v = _cmpex(v, 1 << j, 1 << k, offset)
            return v

        def local_tail(v, k):
            for j in range(log_m_per - 1, -1, -1):
                v = _cmpex(v, 1 << j, 1 << k, offset)
            return v

        def start_exch(grp, e, v):
            _, t = PLAN[e]
            slot = grp * N_EXCH + e
            sbufs[grp] = v
            rdma = pltpu.make_async_remote_copy(
                src_ref=sbufs.at[grp],
                dst_ref=rbufs.at[slot],
                send_sem=send_sems.at[slot],
                recv_sem=recv_sems.at[slot],
                device_id=(my ^ (1 << t),),
                device_id_type=pl.DeviceIdType.MESH,
            )
            if not NO_EXCH:
                rdma.start()
            return rdma

        def finish_exch(grp, e, v, rdma):
            if not NO_EXCH:
                rdma.wait()
            k, t = PLAN[e]
            other = rbufs[grp * N_EXCH + e]
            asc = (offset & (1 << k)) == 0
            partner_above = (offset & (m_per << t)) == 0
            take_min = partner_above == asc
            v = jnp.where(
                take_min, jnp.minimum(v, other), jnp.maximum(v, other)
            )
            if e in TAIL_AFTER:
                v = local_tail(v, TAIL_AFTER[e])
            return v

        vA = x_ref[:, :n_half].astype(jnp.bfloat16)
        vB = x_ref[:, n_half:].astype(jnp.bfloat16)

        vA = local_sort(vA)
        rA = start_exch(0, 0, vA)
        vB = local_sort(vB)
        rB = start_exch(1, 0, vB)
        for e in range(N_EXCH):
            vA = finish_exch(0, e, vA, rA)
            if e + 1 < N_EXCH:
                rA = start_exch(0, e + 1, vA)
            vB = finish_exch(1, e, vB, rB)
            if e + 1 < N_EXCH:
                rB = start_exch(1, e + 1, vB)

        out_ref[:, :n_half] = vA.astype(out_ref.dtype)
        out_ref[:, n_half:] = vB.astype(out_ref.dtype)

    return pl.pallas_call(
        body,
        out_shape=jax.ShapeDtypeStruct((m_per, n), x.dtype),
        in_specs=[pl.BlockSpec(memory_space=pltpu.VMEM)],
        out_specs=pl.BlockSpec(memory_space=pltpu.VMEM),
        scratch_shapes=[
            pltpu.VMEM((2, m_per, n_half), jnp.bfloat16),
            pltpu.VMEM((2 * N_EXCH, m_per, n_half), jnp.bfloat16),
            pltpu.SemaphoreType.DMA((2 * N_EXCH,)),
            pltpu.SemaphoreType.DMA((2 * N_EXCH,)),
        ],
        compiler_params=pltpu.CompilerParams(
            collective_id=0,
            vmem_limit_bytes=100 * 1024 * 1024,
        ),
    )(x)


# device time: 56223 ns/iter; 1.2082x vs baseline; 1.0762x over previous
import os

import jax
import jax.numpy as jnp
from jax import lax
from jax.experimental import pallas as pl
from jax.experimental.pallas import tpu as pltpu

W = 8
LOG_W = 3
PLAN = [(11, 0), (12, 1), (12, 0), (13, 2), (13, 1), (13, 0)]
TAIL_AFTER = {0: 11, 2: 12, 5: 13}
N_EXCH = len(PLAN)
NO_EXCH = bool(os.environ.get("SORT_NO_EXCH"))


def _cmpex(v, s, blk, offset):
    m, n = v.shape
    if s >= 16:
        grp = m // (2 * s)
        y = v.reshape(grp, 2, s, n)
        a, b = y[:, 0], y[:, 1]
        lo, hi = jnp.minimum(a, b), jnp.maximum(a, b)
        g0 = lax.broadcasted_iota(jnp.int32, (grp, 1, 1), 0) * (2 * s) + offset
        asc = (g0 & blk) == 0
        first = jnp.where(asc, lo, hi)
        second = jnp.where(asc, hi, lo)
        out = jnp.concatenate([first[:, None], second[:, None]], axis=1)
        return out.reshape(m, n)
    g = lax.broadcasted_iota(jnp.int32, v.shape, 0) + offset
    partner_above = (g & s) == 0
    down = jnp.concatenate([v[s:], v[:s]], axis=0)
    up = jnp.concatenate([v[m - s:], v[:m - s]], axis=0)
    partner = jnp.where(partner_above, down, up)
    asc = (g & blk) == 0
    take_min = partner_above == asc
    return jnp.where(take_min, jnp.minimum(v, partner), jnp.maximum(v, partner))


def kernel(x):
    m_per, n = x.shape
    n_half = n // 2
    m_half = m_per // 2
    log_m_per = m_per.bit_length() - 1

    def body(x_ref, out_ref, sbufs, rbufs, send_sems, recv_sems):
        my = lax.axis_index("i")
        offset = my * m_per

        barrier_sem = pltpu.get_barrier_semaphore()
        for t in range(LOG_W):
            pl.semaphore_signal(
                barrier_sem, inc=1,
                device_id=(my ^ (1 << t),),
                device_id_type=pl.DeviceIdType.MESH,
            )
        pl.semaphore_wait(barrier_sem, LOG_W)

        def local_sort(v):
            for k in range(1, log_m_per + 1):
                for j in range(k - 1, -1, -1):
                    v = _cmpex(v, 1 << j, 1 << k, offset)
            return v

        def local_tail(v, k):
            for j in range(log_m_per - 1, -1, -1):
                v = _cmpex(v, 1 << j, 1 << k, offset)
            return v

        def start_half(g, e, h, val):
            _, t = PLAN[e]
            slot = (g * N_EXCH + e) * 2 + h
            sbufs[g, h] = val
            rdma = pltpu.make_async_remote_copy(
                src_ref=sbufs.at[g, h],
                dst_ref=rbufs.at[slot],
                send_sem=send_sems.at[slot],
                recv_sem=recv_sems.at[slot],
                device_id=(my ^ (1 << t),),
                device_id_type=pl.DeviceIdType.MESH,
            )
            if not NO_EXCH:
                rdma.start()
            return rdma

        def finish_half(g, e, h, val, rdma):
            if not NO_EXCH:
                rdma.wait()
            k, t = PLAN[e]
            other = rbufs[(g * N_EXCH + e) * 2 + h]
            asc = (offset & (1 << k)) == 0
            partner_above = (offset & (m_per << t)) == 0
            take_min = partner_above == asc
            return jnp.where(
                take_min, jnp.minimum(val, other), jnp.maximum(val, other)
            )

        halves = [[None, None], [None, None]]
        rd = [[None, None], [None, None]]
        vA = local_sort(x_ref[:, :n_half].astype(jnp.bfloat16))
        halves[0] = [vA[:m_half], vA[m_half:]]
        for h in (0, 1):
            rd[0][h] = start_half(0, 0, h, halves[0][h])
        vB = local_sort(x_ref[:, n_half:].astype(jnp.bfloat16))
        halves[1] = [vB[:m_half], vB[m_half:]]
        for h in (0, 1):
            rd[1][h] = start_half(1, 0, h, halves[1][h])

        for e in range(N_EXCH):
            nxt = e + 1
            for g in (0, 1):
                if e in TAIL_AFTER:
                    for h in (0, 1):
                        halves[g][h] = finish_half(g, e, h, halves[g][h], rd[g][h])
                    v = jnp.concatenate(halves[g], axis=0)
                    v = local_tail(v, TAIL_AFTER[e])
                    if nxt < N_EXCH:
                        halves[g] = [v[:m_half], v[m_half:]]
                        for h in (0, 1):
                            rd[g][h] = start_half(g, nxt, h, halves[g][h])
                    else:
                        cols = pl.ds(g * n_half, n_half)
                        out_ref[:, cols] = v.astype(out_ref.dtype)
                else:
                    for h in (0, 1):
                        halves[g][h] = finish_half(g, e, h, halves[g][h], rd[g][h])
                        rd[g][h] = start_half(g, nxt, h, halves[g][h])

    return pl.pallas_call(
        body,
        out_shape=jax.ShapeDtypeStruct((m_per, n), x.dtype),
        in_specs=[pl.BlockSpec(memory_space=pltpu.VMEM)],
        out_specs=pl.BlockSpec(memory_space=pltpu.VMEM),
        scratch_shapes=[
            pltpu.VMEM((2, 2, m_half, n_half), jnp.bfloat16),
            pltpu.VMEM((2 * N_EXCH * 2, m_half, n_half), jnp.bfloat16),
            pltpu.SemaphoreType.DMA((2 * N_EXCH * 2,)),
            pltpu.SemaphoreType.DMA((2 * N_EXCH * 2,)),
        ],
        compiler_params=pltpu.CompilerParams(
            collective_id=0,
            vmem_limit_bytes=100 * 1024 * 1024,
        ),
    )(x)


# device time: 53410 ns/iter; 1.2719x vs baseline; 1.0527x over previous
import os

import jax
import jax.numpy as jnp
from jax import lax
from jax.experimental import pallas as pl
from jax.experimental.pallas import tpu as pltpu

W = 8
LOG_W = 3
PLAN = [(11, 0), (12, 1), (12, 0), (13, 2), (13, 1), (13, 0)]
TAIL_AFTER = {0: 11, 2: 12, 5: 13}
N_EXCH = len(PLAN)
N_CHUNK = int(os.environ.get("SORT_CHUNKS", "4"))
NO_EXCH = bool(os.environ.get("SORT_NO_EXCH"))


def _cmpex(v, s, blk, offset):
    m, n = v.shape
    if s >= 16:
        grp = m // (2 * s)
        y = v.reshape(grp, 2, s, n)
        a, b = y[:, 0], y[:, 1]
        lo, hi = jnp.minimum(a, b), jnp.maximum(a, b)
        g0 = lax.broadcasted_iota(jnp.int32, (grp, 1, 1), 0) * (2 * s) + offset
        asc = (g0 & blk) == 0
        first = jnp.where(asc, lo, hi)
        second = jnp.where(asc, hi, lo)
        out = jnp.concatenate([first[:, None], second[:, None]], axis=1)
        return out.reshape(m, n)
    g = lax.broadcasted_iota(jnp.int32, v.shape, 0) + offset
    partner_above = (g & s) == 0
    down = jnp.concatenate([v[s:], v[:s]], axis=0)
    up = jnp.concatenate([v[m - s:], v[:m - s]], axis=0)
    partner = jnp.where(partner_above, down, up)
    asc = (g & blk) == 0
    take_min = partner_above == asc
    return jnp.where(take_min, jnp.minimum(v, partner), jnp.maximum(v, partner))


def kernel(x):
    m_per, n = x.shape
    n_half = n // 2
    m_chunk = m_per // N_CHUNK
    log_m_per = m_per.bit_length() - 1

    def body(x_ref, out_ref, sbufs, rbufs, send_sems, recv_sems):
        my = lax.axis_index("i")
        offset = my * m_per

        barrier_sem = pltpu.get_barrier_semaphore()
        for t in range(LOG_W):
            pl.semaphore_signal(
                barrier_sem, inc=1,
                device_id=(my ^ (1 << t),),
                device_id_type=pl.DeviceIdType.MESH,
            )
        pl.semaphore_wait(barrier_sem, LOG_W)

        def local_sort(v):
            for k in range(1, log_m_per + 1):
                for j in range(k - 1, -1, -1):
                    v = _cmpex(v, 1 << j, 1 << k, offset)
            return v

        def local_tail(v, k):
            for j in range(log_m_per - 1, -1, -1):
                v = _cmpex(v, 1 << j, 1 << k, offset)
            return v

        def start_half(g, e, h, val):
            _, t = PLAN[e]
            slot = (g * N_EXCH + e) * N_CHUNK + h
            sbufs[g, h] = val
            rdma = pltpu.make_async_remote_copy(
                src_ref=sbufs.at[g, h],
                dst_ref=rbufs.at[slot],
                send_sem=send_sems.at[slot],
                recv_sem=recv_sems.at[slot],
                device_id=(my ^ (1 << t),),
                device_id_type=pl.DeviceIdType.MESH,
            )
            if not NO_EXCH:
                rdma.start()
            return rdma

        def finish_half(g, e, h, val, rdma):
            if not NO_EXCH:
                rdma.wait()
            k, t = PLAN[e]
            other = rbufs[(g * N_EXCH + e) * N_CHUNK + h]
            asc = (offset & (1 << k)) == 0
            partner_above = (offset & (m_per << t)) == 0
            take_min = partner_above == asc
            return jnp.where(
                take_min, jnp.minimum(val, other), jnp.maximum(val, other)
            )

        def split(v):
            return [v[c * m_chunk:(c + 1) * m_chunk] for c in range(N_CHUNK)]

        chunks = [None, None]
        rd = [[None] * N_CHUNK, [None] * N_CHUNK]
        vA = local_sort(x_ref[:, :n_half].astype(jnp.bfloat16))
        chunks[0] = split(vA)
        for h in range(N_CHUNK):
            rd[0][h] = start_half(0, 0, h, chunks[0][h])
        vB = local_sort(x_ref[:, n_half:].astype(jnp.bfloat16))
        chunks[1] = split(vB)
        for h in range(N_CHUNK):
            rd[1][h] = start_half(1, 0, h, chunks[1][h])

        for e in range(N_EXCH):
            nxt = e + 1
            for g in (0, 1):
                if e in TAIL_AFTER:
                    for h in range(N_CHUNK):
                        chunks[g][h] = finish_half(g, e, h, chunks[g][h], rd[g][h])
                    v = jnp.concatenate(chunks[g], axis=0)
                    v = local_tail(v, TAIL_AFTER[e])
                    if nxt < N_EXCH:
                        chunks[g] = split(v)
                        for h in range(N_CHUNK):
                            rd[g][h] = start_half(g, nxt, h, chunks[g][h])
                    else:
                        cols = pl.ds(g * n_half, n_half)
                        out_ref[:, cols] = v.astype(out_ref.dtype)
                else:
                    for h in range(N_CHUNK):
                        chunks[g][h] = finish_half(g, e, h, chunks[g][h], rd[g][h])
                        rd[g][h] = start_half(g, nxt, h, chunks[g][h])

    return pl.pallas_call(
        body,
        out_shape=jax.ShapeDtypeStruct((m_per, n), x.dtype),
        in_specs=[pl.BlockSpec(memory_space=pltpu.VMEM)],
        out_specs=pl.BlockSpec(memory_space=pltpu.VMEM),
        scratch_shapes=[
            pltpu.VMEM((2, N_CHUNK, m_chunk, n_half), jnp.bfloat16),
            pltpu.VMEM((2 * N_EXCH * N_CHUNK, m_chunk, n_half), jnp.bfloat16),
            pltpu.SemaphoreType.DMA((2 * N_EXCH * N_CHUNK,)),
            pltpu.SemaphoreType.DMA((2 * N_EXCH * N_CHUNK,)),
        ],
        compiler_params=pltpu.CompilerParams(
            collective_id=0,
            vmem_limit_bytes=100 * 1024 * 1024,
        ),
    )(x)
